# baseline (speedup 1.0000x reference)
import functools

import jax
import jax.numpy as jnp
from jax import lax
from jax.experimental import pallas as pl
from jax.experimental.pallas import tpu as pltpu
from jax.experimental.pallas import tpu_sc as plsc

NN = 10000
NP = 10240
EE = 320000
FD = 128
NC = 2
NS = 16
NW = NC * NS
CH = 80
EPA = 327680
EPP = 647168
AIR = 80
PIR = 158
NPW = NP // NS

f32 = jnp.float32


def _mesh():
    return plsc.VectorSubcoreMesh(core_axis_name="c", subcore_axis_name="s")


def _hist_body(src_r, dst_r, ones_h, zeros1_h, out_deg, idx_v, ones_v, acc_sh):
    c = lax.axis_index("c")
    s = lax.axis_index("s")
    rows_per_tile = EE // CH // NS
    pltpu.sync_copy(zeros1_h.at[pl.ds(s * NPW, NPW)],
                    acc_sh.at[pl.ds(s * NPW, NPW)])
    pltpu.sync_copy(ones_h, ones_v)

    @pl.when(c == 0)
    def _():
        pltpu.sync_copy(src_r.at[s], idx_v)

    @pl.when(c == 1)
    def _():
        pltpu.sync_copy(dst_r.at[s], idx_v)

    plsc.subcore_barrier()

    def chunk(j, carry):
        pltpu.sync_copy(ones_v, acc_sh.at[idx_v.at[j]], add=True)
        return carry

    lax.fori_loop(0, rows_per_tile, chunk, 0)
    plsc.subcore_barrier()
    pltpu.sync_copy(acc_sh.at[pl.ds(s * NPW, NPW)],
                    out_deg.at[pl.ds(c * NP + s * NPW, NPW)])


@functools.partial(
    pl.kernel,
    out_type=jax.ShapeDtypeStruct((2 * NP,), f32),
    mesh=_mesh(),
    scratch_types=[
        pltpu.VMEM((EE // CH // NS, CH), jnp.int32),
        pltpu.VMEM((CH,), f32),
        pltpu.VMEM_SHARED((NP,), f32),
    ],
)
def _hist(src_r, dst_r, ones_h, zeros1_h, out_deg, idx_v, ones_v, acc_sh):
    _hist_body(src_r, dst_r, ones_h, zeros1_h, out_deg, idx_v, ones_v, acc_sh)


def _agg_body(y_h, src_r, dst_r, zeros2_h, out_p,
              isv, idv, rows_v, acc_sh, gsem, ssem):
    c = lax.axis_index("c")
    s = lax.axis_index("s")
    wid = c * NS + s
    nchunks = 4 * AIR
    pltpu.sync_copy(zeros2_h.at[pl.ds(s * NPW, NPW)],
                    acc_sh.at[pl.ds(s * NPW, NPW)])
    pltpu.sync_copy(src_r.at[wid], isv)
    pltpu.sync_copy(dst_r.at[wid], idv)
    plsc.subcore_barrier()

    def _sidx(ref, t):
        return ref.at[t // 4, pl.ds((t % 4) * 32, 32)]

    for b in range(2):
        pltpu.make_async_copy(y_h.at[_sidx(isv, b)], rows_v.at[b],
                              gsem.at[b]).start()

    @pl.loop(0, nchunks, step=4)
    def _outer(tt):
        for b in range(4):
            t = tt + b
            pltpu.make_async_copy(y_h.at[_sidx(isv, b)], rows_v.at[b],
                                  gsem.at[b]).wait()
            pltpu.make_async_copy(rows_v.at[b], acc_sh.at[_sidx(idv, t)],
                                  ssem.at[b]).start(add=True)
            bq = (b + 2) % 4

            @pl.when(t + 2 < nchunks)
            def _():
                @pl.when(t >= 2)
                def _():
                    pltpu.make_async_copy(rows_v.at[bq],
                                          acc_sh.at[_sidx(idv, 0)],
                                          ssem.at[bq]).wait()

                pltpu.make_async_copy(y_h.at[_sidx(isv, t + 2)],
                                      rows_v.at[bq], gsem.at[bq]).start()

    for b in range(4):
        pltpu.make_async_copy(rows_v.at[b], acc_sh.at[_sidx(idv, 0)],
                              ssem.at[b]).wait()
    plsc.subcore_barrier()
    pltpu.sync_copy(acc_sh.at[pl.ds(s * NPW, NPW)],
                    out_p.at[pl.ds(c * NP + s * NPW, NPW)])


@functools.partial(
    pl.kernel,
    out_type=jax.ShapeDtypeStruct((2 * NP, FD), f32),
    mesh=_mesh(),
    scratch_types=[
        pltpu.VMEM((AIR, 128), jnp.int32),
        pltpu.VMEM((AIR, 128), jnp.int32),
        pltpu.VMEM((4, 32, FD), f32),
        pltpu.VMEM_SHARED((NP, FD), f32),
        pltpu.SemaphoreType.DMA((4,)),
        pltpu.SemaphoreType.DMA((4,)),
    ],
)
def _agg(y_h, src_r, dst_r, zeros2_h, out_p,
         isv, idv, rows_v, acc_sh, gsem, ssem):
    _agg_body(y_h, src_r, dst_r, zeros2_h, out_p,
              isv, idv, rows_v, acc_sh, gsem, ssem)


def _pred_body(u_h, v_h, src_r, dst_r, par_h, bpb_h, out_s,
               isv, idv, ru, rv, par_v, bpb_v, acc_buf, stage, gsem, osem):
    c = lax.axis_index("c")
    s = lax.axis_index("s")
    wid = c * NS + s
    base = wid * PIR * 128
    pltpu.sync_copy(src_r.at[wid], isv)
    pltpu.sync_copy(dst_r.at[wid], idv)
    pltpu.sync_copy(par_h, par_v)
    pltpu.sync_copy(bpb_h, bpb_v)
    w2r = [par_v[0, pl.ds(i * 16, 16)] for i in range(8)]
    biasp = [bpb_v[0, pl.ds(i * 16, 16)] for i in range(8)]
    acc0 = par_v[1, pl.ds(0, 16)]

    for b in range(2):
        pltpu.make_async_copy(u_h.at[isv.at[b]], ru.at[b], gsem.at[b]).start()
        pltpu.make_async_copy(v_h.at[idv.at[b]], rv.at[b], gsem.at[b]).start()

    @pl.loop(0, PIR, step=2)
    def _outer(tt):
        for b in range(2):
            t = tt + b
            pltpu.make_async_copy(u_h.at[isv.at[b]], ru.at[b],
                                  gsem.at[b]).wait()
            pltpu.make_async_copy(v_h.at[idv.at[b]], rv.at[b],
                                  gsem.at[b]).wait()

            @pl.when(t >= 2)
            def _():
                pltpu.make_async_copy(stage.at[b], out_s.at[pl.ds(0, 128)],
                                      osem.at[b]).wait()

            def edge(e, c2):
                acc = acc0
                for k in range(FD // 16):
                    sl = pl.ds(k * 16, 16)
                    a = jnp.maximum(ru[b, e, sl] + rv[b, e, sl] + biasp[k],
                                    0.0)
                    acc = acc + a * w2r[k]
                acc_buf[pl.ds(e * 16, 16)] = acc
                return c2

            lax.fori_loop(0, 128, edge, 0, unroll=8)

            def grp(g, c2):
                ridx = (g * 16 + lax.iota(jnp.int32, 16)) * 16
                tot = plsc.load_gather(acc_buf, [ridx])
                for kk in range(1, 16):
                    tot = tot + plsc.load_gather(acc_buf, [ridx + kk])
                stage[b, pl.ds(g * 16, 16)] = tot
                return c2

            lax.fori_loop(0, 8, grp, 0, unroll=True)

            pltpu.make_async_copy(stage.at[b],
                                  out_s.at[pl.ds(base + t * 128, 128)],
                                  osem.at[b]).start()

            @pl.when(t + 2 < PIR)
            def _():
                pltpu.make_async_copy(u_h.at[isv.at[t + 2]], ru.at[b],
                                      gsem.at[b]).start()
                pltpu.make_async_copy(v_h.at[idv.at[t + 2]], rv.at[b],
                                      gsem.at[b]).start()

    for b in range(2):
        pltpu.make_async_copy(stage.at[b], out_s.at[pl.ds(0, 128)],
                              osem.at[b]).wait()


@functools.partial(
    pl.kernel,
    out_type=jax.ShapeDtypeStruct((EPP,), f32),
    mesh=_mesh(),
    compiler_params=pltpu.CompilerParams(needs_layout_passes=False),
    scratch_types=[
        pltpu.VMEM((PIR, 128), jnp.int32),
        pltpu.VMEM((PIR, 128), jnp.int32),
        pltpu.VMEM((2, 128, FD), f32),
        pltpu.VMEM((2, 128, FD), f32),
        pltpu.VMEM((2, FD), f32),
        pltpu.VMEM((1, FD), f32),
        pltpu.VMEM((128 * 16,), f32),
        pltpu.VMEM((2, 128), f32),
        pltpu.SemaphoreType.DMA((2,)),
        pltpu.SemaphoreType.DMA((2,)),
    ],
)
def _pred(u_h, v_h, src_r, dst_r, par_h, bpb_h, out_s,
          isv, idv, ru, rv, par_v, bpb_v, acc_buf, stage, gsem, osem):
    _pred_body(u_h, v_h, src_r, dst_r, par_h, bpb_h, out_s,
               isv, idv, ru, rv, par_v, bpb_v, acc_buf, stage, gsem, osem)


RB = 2048


def _tc1_body(x_ref, deg_ref, w_ref, y_ref):
    i = pl.program_id(0)
    ns = lax.rsqrt(jnp.maximum(deg_ref[0, pl.ds(i * RB, RB)], 1.0))
    xs = x_ref[...] * ns[:, None]
    y_ref[...] = jnp.dot(xs, w_ref[...], preferred_element_type=f32,
                       precision=lax.Precision.HIGHEST)


def _tc1(xp, deg2, W1):
    return pl.pallas_call(
        _tc1_body,
        grid=(NP // RB,),
        in_specs=[
            pl.BlockSpec((RB, FD), lambda i: (i, 0)),
            pl.BlockSpec((2, NP), lambda i: (0, 0)),
            pl.BlockSpec((FD, FD), lambda i: (0, 0)),
        ],
        out_specs=pl.BlockSpec((RB, FD), lambda i: (i, 0)),
        out_shape=jax.ShapeDtypeStruct((NP, FD), f32),
    )(xp, deg2, W1)


def _tc2_body(p_ref, deg_ref, w_ref, y_ref):
    i = pl.program_id(0)
    ns = lax.rsqrt(jnp.maximum(deg_ref[0, pl.ds(i * RB, RB)], 1.0))
    nd = lax.rsqrt(jnp.maximum(deg_ref[1, pl.ds(i * RB, RB)], 1.0))
    a = p_ref[0] + p_ref[1]
    h1 = jnp.maximum(a * nd[:, None], 0.0)
    y_ref[...] = jnp.dot(h1 * ns[:, None], w_ref[...], preferred_element_type=f32,
                       precision=lax.Precision.HIGHEST)


def _tc2(p1, deg2, W2):
    return pl.pallas_call(
        _tc2_body,
        grid=(NP // RB,),
        in_specs=[
            pl.BlockSpec((2, RB, FD), lambda i: (0, i, 0)),
            pl.BlockSpec((2, NP), lambda i: (0, 0)),
            pl.BlockSpec((FD, FD), lambda i: (0, 0)),
        ],
        out_specs=pl.BlockSpec((RB, FD), lambda i: (i, 0)),
        out_shape=jax.ShapeDtypeStruct((NP, FD), f32),
    )(p1, deg2, W2)


def _tc3_body(p_ref, deg_ref, wp_ref, h_ref, u_ref, v_ref):
    i = pl.program_id(0)
    nd = lax.rsqrt(jnp.maximum(deg_ref[1, pl.ds(i * RB, RB)], 1.0))
    h = (p_ref[0] + p_ref[1]) * nd[:, None]
    h_ref[...] = h
    u_ref[...] = jnp.dot(h, wp_ref[0:FD, :], preferred_element_type=f32,
                         precision=lax.Precision.HIGHEST)
    v_ref[...] = jnp.dot(h, wp_ref[FD:2 * FD, :], preferred_element_type=f32,
                         precision=lax.Precision.HIGHEST)


def _tc3(p2, deg2, Wp1):
    return pl.pallas_call(
        _tc3_body,
        grid=(NP // RB,),
        in_specs=[
            pl.BlockSpec((2, RB, FD), lambda i: (0, i, 0)),
            pl.BlockSpec((2, NP), lambda i: (0, 0)),
            pl.BlockSpec((2 * FD, FD), lambda i: (0, 0)),
        ],
        out_specs=[
            pl.BlockSpec((RB, FD), lambda i: (i, 0)),
            pl.BlockSpec((RB, FD), lambda i: (i, 0)),
            pl.BlockSpec((RB, FD), lambda i: (i, 0)),
        ],
        out_shape=[
            jax.ShapeDtypeStruct((NP, FD), f32),
            jax.ShapeDtypeStruct((NP, FD), f32),
            jax.ShapeDtypeStruct((NP, FD), f32),
        ],
    )(p2, deg2, Wp1)


def kernel(x, edge_index, neg_edge_index, W1, W2, Wp1, bp1, Wp2, bp2):
    src, dst = edge_index[0], edge_index[1]
    src_h = src.reshape(NS, EE // CH // NS, CH)
    dst_h = dst.reshape(NS, EE // CH // NS, CH)
    epad = NN + jnp.arange(EPA - EE, dtype=jnp.int32) % (NP - NN)
    src_r = jnp.concatenate([src, epad]).reshape(NW, AIR, 128)
    dst_r = jnp.concatenate([dst, epad]).reshape(NW, AIR, 128)
    xp = jnp.pad(x, ((0, NP - NN), (0, 0)))
    ones = jnp.ones((CH,), f32)
    zeros1 = jnp.zeros((NP,), f32)
    zeros2 = jnp.zeros((NP, FD), f32)

    deg2 = _hist(src_h, dst_h, ones, zeros1).reshape(2, NP)
    y1 = _tc1(xp, deg2, W1)
    p1 = _agg(y1, src_r, dst_r, zeros2).reshape(2, NP, FD)
    y2 = _tc2(p1, deg2, W2)
    p2 = _agg(y2, src_r, dst_r, zeros2).reshape(2, NP, FD)
    h, u, v = _tc3(p2, deg2, Wp1)

    epad2 = NN + jnp.arange(EPP - 2 * EE, dtype=jnp.int32) % (NP - NN)
    src2_r = jnp.concatenate([src, neg_edge_index[0], epad2]).reshape(
        NW, PIR, 128)
    dst2_r = jnp.concatenate([dst, neg_edge_index[1], epad2]).reshape(
        NW, PIR, 128)
    par = jnp.stack([Wp2[:, 0], jnp.zeros((FD,), f32).at[0].set(bp2[0])])
    bpb = bp1[None, :]
    scores = _pred(u, v, src2_r, dst2_r, par, bpb)
    return scores[:EE, None], scores[EE:2 * EE, None], h[:NN]


# speedup vs baseline: 1.0624x; 1.0624x over previous
import functools

import jax
import jax.numpy as jnp
from jax import lax
from jax.experimental import pallas as pl
from jax.experimental.pallas import tpu as pltpu
from jax.experimental.pallas import tpu_sc as plsc

NN = 10000
NP = 10240
EE = 320000
FD = 128
NC = 2
NS = 16
NW = NC * NS
CH = 80
EPA = 327680
EPP = 647168
AIR = 80
PIR = 158
NPW = NP // NS

f32 = jnp.float32


def _mesh():
    return plsc.VectorSubcoreMesh(core_axis_name="c", subcore_axis_name="s")


def _hist_body(src_r, dst_r, ones_h, zeros1_h, out_deg, idx_v, ones_v, acc_sh):
    c = lax.axis_index("c")
    s = lax.axis_index("s")
    rows_per_tile = EE // CH // NS
    pltpu.sync_copy(zeros1_h.at[pl.ds(s * NPW, NPW)],
                    acc_sh.at[pl.ds(s * NPW, NPW)])
    pltpu.sync_copy(ones_h, ones_v)

    @pl.when(c == 0)
    def _():
        pltpu.sync_copy(src_r.at[s], idx_v)

    @pl.when(c == 1)
    def _():
        pltpu.sync_copy(dst_r.at[s], idx_v)

    plsc.subcore_barrier()

    def chunk(j, carry):
        pltpu.sync_copy(ones_v, acc_sh.at[idx_v.at[j]], add=True)
        return carry

    lax.fori_loop(0, rows_per_tile, chunk, 0)
    plsc.subcore_barrier()
    pltpu.sync_copy(acc_sh.at[pl.ds(s * NPW, NPW)],
                    out_deg.at[pl.ds(c * NP + s * NPW, NPW)])


@functools.partial(
    pl.kernel,
    out_type=jax.ShapeDtypeStruct((2 * NP,), f32),
    mesh=_mesh(),
    scratch_types=[
        pltpu.VMEM((EE // CH // NS, CH), jnp.int32),
        pltpu.VMEM((CH,), f32),
        pltpu.VMEM_SHARED((NP,), f32),
    ],
)
def _hist(src_r, dst_r, ones_h, zeros1_h, out_deg, idx_v, ones_v, acc_sh):
    _hist_body(src_r, dst_r, ones_h, zeros1_h, out_deg, idx_v, ones_v, acc_sh)


def _agg_body(y_h, src_r, dst_r, zeros2_h, out_p,
              isv, idv, rows_v, acc_sh, gsem, ssem):
    c = lax.axis_index("c")
    s = lax.axis_index("s")
    wid = c * NS + s
    nchunks = 2 * AIR
    pltpu.sync_copy(zeros2_h.at[pl.ds(s * NPW, NPW)],
                    acc_sh.at[pl.ds(s * NPW, NPW)])
    pltpu.sync_copy(src_r.at[wid], isv)
    pltpu.sync_copy(dst_r.at[wid], idv)
    plsc.subcore_barrier()

    def _sidx(ref, t):
        return ref.at[t // 2, pl.ds((t % 2) * 64, 64)]

    for b in range(2):
        pltpu.make_async_copy(y_h.at[_sidx(isv, b)], rows_v.at[b],
                              gsem.at[b]).start()

    @pl.loop(0, nchunks, step=2)
    def _outer(tt):
        for b in range(2):
            t = tt + b
            pltpu.make_async_copy(y_h.at[_sidx(isv, b)], rows_v.at[b],
                                  gsem.at[b]).wait()
            sc = pltpu.make_async_copy(rows_v.at[b], acc_sh.at[_sidx(idv, t)],
                                       ssem.at[b])
            sc.start(add=True)
            sc.wait()

            @pl.when(t + 2 < nchunks)
            def _():
                pltpu.make_async_copy(y_h.at[_sidx(isv, t + 2)], rows_v.at[b],
                                      gsem.at[b]).start()

    plsc.subcore_barrier()
    pltpu.sync_copy(acc_sh.at[pl.ds(s * NPW, NPW)],
                    out_p.at[pl.ds(c * NP + s * NPW, NPW)])


@functools.partial(
    pl.kernel,
    out_type=jax.ShapeDtypeStruct((2 * NP, FD), f32),
    mesh=_mesh(),
    scratch_types=[
        pltpu.VMEM((AIR, 128), jnp.int32),
        pltpu.VMEM((AIR, 128), jnp.int32),
        pltpu.VMEM((2, 64, FD), f32),
        pltpu.VMEM_SHARED((NP, FD), f32),
        pltpu.SemaphoreType.DMA((2,)),
        pltpu.SemaphoreType.DMA((2,)),
    ],
)
def _agg(y_h, src_r, dst_r, zeros2_h, out_p,
         isv, idv, rows_v, acc_sh, gsem, ssem):
    _agg_body(y_h, src_r, dst_r, zeros2_h, out_p,
              isv, idv, rows_v, acc_sh, gsem, ssem)


def _pred_body(u_h, v_h, src_r, dst_r, par_h, bpb_h, out_s,
               isv, idv, ru, rv, par_v, bpb_v, acc_buf, stage, gsem, osem):
    c = lax.axis_index("c")
    s = lax.axis_index("s")
    wid = c * NS + s
    base = wid * PIR * 128
    pltpu.sync_copy(src_r.at[wid], isv)
    pltpu.sync_copy(dst_r.at[wid], idv)
    pltpu.sync_copy(par_h, par_v)
    pltpu.sync_copy(bpb_h, bpb_v)
    w2r = [par_v[0, pl.ds(i * 16, 16)] for i in range(8)]
    biasp = [bpb_v[0, pl.ds(i * 16, 16)] for i in range(8)]
    acc0 = par_v[1, pl.ds(0, 16)]

    for b in range(2):
        pltpu.make_async_copy(u_h.at[isv.at[b]], ru.at[b], gsem.at[b]).start()
        pltpu.make_async_copy(v_h.at[idv.at[b]], rv.at[b], gsem.at[b]).start()

    @pl.loop(0, PIR, step=2)
    def _outer(tt):
        for b in range(2):
            t = tt + b
            pltpu.make_async_copy(u_h.at[isv.at[b]], ru.at[b],
                                  gsem.at[b]).wait()
            pltpu.make_async_copy(v_h.at[idv.at[b]], rv.at[b],
                                  gsem.at[b]).wait()

            @pl.when(t >= 2)
            def _():
                pltpu.make_async_copy(stage.at[b], out_s.at[pl.ds(0, 128)],
                                      osem.at[b]).wait()

            def edge(e, c2):
                acc = acc0
                for k in range(FD // 16):
                    sl = pl.ds(k * 16, 16)
                    a = jnp.maximum(ru[b, e, sl] + rv[b, e, sl] + biasp[k],
                                    0.0)
                    acc = acc + a * w2r[k]
                acc_buf[pl.ds(e * 16, 16)] = acc
                return c2

            lax.fori_loop(0, 128, edge, 0, unroll=8)

            def grp(g, c2):
                ridx = (g * 16 + lax.iota(jnp.int32, 16)) * 16
                tot = plsc.load_gather(acc_buf, [ridx])
                for kk in range(1, 16):
                    tot = tot + plsc.load_gather(acc_buf, [ridx + kk])
                stage[b, pl.ds(g * 16, 16)] = tot
                return c2

            lax.fori_loop(0, 8, grp, 0, unroll=True)

            pltpu.make_async_copy(stage.at[b],
                                  out_s.at[pl.ds(base + t * 128, 128)],
                                  osem.at[b]).start()

            @pl.when(t + 2 < PIR)
            def _():
                pltpu.make_async_copy(u_h.at[isv.at[t + 2]], ru.at[b],
                                      gsem.at[b]).start()
                pltpu.make_async_copy(v_h.at[idv.at[t + 2]], rv.at[b],
                                      gsem.at[b]).start()

    for b in range(2):
        pltpu.make_async_copy(stage.at[b], out_s.at[pl.ds(0, 128)],
                              osem.at[b]).wait()


@functools.partial(
    pl.kernel,
    out_type=jax.ShapeDtypeStruct((EPP,), f32),
    mesh=_mesh(),
    compiler_params=pltpu.CompilerParams(needs_layout_passes=False),
    scratch_types=[
        pltpu.VMEM((PIR, 128), jnp.int32),
        pltpu.VMEM((PIR, 128), jnp.int32),
        pltpu.VMEM((2, 128, FD), f32),
        pltpu.VMEM((2, 128, FD), f32),
        pltpu.VMEM((2, FD), f32),
        pltpu.VMEM((1, FD), f32),
        pltpu.VMEM((128 * 16,), f32),
        pltpu.VMEM((2, 128), f32),
        pltpu.SemaphoreType.DMA((2,)),
        pltpu.SemaphoreType.DMA((2,)),
    ],
)
def _pred(u_h, v_h, src_r, dst_r, par_h, bpb_h, out_s,
          isv, idv, ru, rv, par_v, bpb_v, acc_buf, stage, gsem, osem):
    _pred_body(u_h, v_h, src_r, dst_r, par_h, bpb_h, out_s,
               isv, idv, ru, rv, par_v, bpb_v, acc_buf, stage, gsem, osem)


RB = 2048


def _tc1_body(x_ref, deg_ref, w_ref, y_ref):
    i = pl.program_id(0)
    ns = lax.rsqrt(jnp.maximum(deg_ref[0, pl.ds(i * RB, RB)], 1.0))
    xs = x_ref[...] * ns[:, None]
    y_ref[...] = jnp.dot(xs, w_ref[...], preferred_element_type=f32,
                       precision=lax.Precision.HIGHEST)


def _tc1(xp, deg2, W1):
    return pl.pallas_call(
        _tc1_body,
        grid=(NP // RB,),
        in_specs=[
            pl.BlockSpec((RB, FD), lambda i: (i, 0)),
            pl.BlockSpec((2, NP), lambda i: (0, 0)),
            pl.BlockSpec((FD, FD), lambda i: (0, 0)),
        ],
        out_specs=pl.BlockSpec((RB, FD), lambda i: (i, 0)),
        out_shape=jax.ShapeDtypeStruct((NP, FD), f32),
    )(xp, deg2, W1)


def _tc2_body(p_ref, deg_ref, w_ref, y_ref):
    i = pl.program_id(0)
    ns = lax.rsqrt(jnp.maximum(deg_ref[0, pl.ds(i * RB, RB)], 1.0))
    nd = lax.rsqrt(jnp.maximum(deg_ref[1, pl.ds(i * RB, RB)], 1.0))
    a = p_ref[0] + p_ref[1]
    h1 = jnp.maximum(a * nd[:, None], 0.0)
    y_ref[...] = jnp.dot(h1 * ns[:, None], w_ref[...], preferred_element_type=f32,
                       precision=lax.Precision.HIGHEST)


def _tc2(p1, deg2, W2):
    return pl.pallas_call(
        _tc2_body,
        grid=(NP // RB,),
        in_specs=[
            pl.BlockSpec((2, RB, FD), lambda i: (0, i, 0)),
            pl.BlockSpec((2, NP), lambda i: (0, 0)),
            pl.BlockSpec((FD, FD), lambda i: (0, 0)),
        ],
        out_specs=pl.BlockSpec((RB, FD), lambda i: (i, 0)),
        out_shape=jax.ShapeDtypeStruct((NP, FD), f32),
    )(p1, deg2, W2)


def _tc3_body(p_ref, deg_ref, wp_ref, h_ref, u_ref, v_ref):
    i = pl.program_id(0)
    nd = lax.rsqrt(jnp.maximum(deg_ref[1, pl.ds(i * RB, RB)], 1.0))
    h = (p_ref[0] + p_ref[1]) * nd[:, None]
    h_ref[...] = h
    u_ref[...] = jnp.dot(h, wp_ref[0:FD, :], preferred_element_type=f32,
                         precision=lax.Precision.HIGHEST)
    v_ref[...] = jnp.dot(h, wp_ref[FD:2 * FD, :], preferred_element_type=f32,
                         precision=lax.Precision.HIGHEST)


def _tc3(p2, deg2, Wp1):
    return pl.pallas_call(
        _tc3_body,
        grid=(NP // RB,),
        in_specs=[
            pl.BlockSpec((2, RB, FD), lambda i: (0, i, 0)),
            pl.BlockSpec((2, NP), lambda i: (0, 0)),
            pl.BlockSpec((2 * FD, FD), lambda i: (0, 0)),
        ],
        out_specs=[
            pl.BlockSpec((RB, FD), lambda i: (i, 0)),
            pl.BlockSpec((RB, FD), lambda i: (i, 0)),
            pl.BlockSpec((RB, FD), lambda i: (i, 0)),
        ],
        out_shape=[
            jax.ShapeDtypeStruct((NP, FD), f32),
            jax.ShapeDtypeStruct((NP, FD), f32),
            jax.ShapeDtypeStruct((NP, FD), f32),
        ],
    )(p2, deg2, Wp1)


def kernel(x, edge_index, neg_edge_index, W1, W2, Wp1, bp1, Wp2, bp2):
    src, dst = edge_index[0], edge_index[1]
    src_h = src.reshape(NS, EE // CH // NS, CH)
    dst_h = dst.reshape(NS, EE // CH // NS, CH)
    epad = NN + jnp.arange(EPA - EE, dtype=jnp.int32) % (NP - NN)
    src_r = jnp.concatenate([src, epad]).reshape(NW, AIR, 128)
    dst_r = jnp.concatenate([dst, epad]).reshape(NW, AIR, 128)
    xp = jnp.pad(x, ((0, NP - NN), (0, 0)))
    ones = jnp.ones((CH,), f32)
    zeros1 = jnp.zeros((NP,), f32)
    zeros2 = jnp.zeros((NP, FD), f32)

    deg2 = _hist(src_h, dst_h, ones, zeros1).reshape(2, NP)
    y1 = _tc1(xp, deg2, W1)
    p1 = _agg(y1, src_r, dst_r, zeros2).reshape(2, NP, FD)
    y2 = _tc2(p1, deg2, W2)
    p2 = _agg(y2, src_r, dst_r, zeros2).reshape(2, NP, FD)
    h, u, v = _tc3(p2, deg2, Wp1)

    epad2 = NN + jnp.arange(EPP - 2 * EE, dtype=jnp.int32) % (NP - NN)
    src2_r = jnp.concatenate([src, neg_edge_index[0], epad2]).reshape(
        NW, PIR, 128)
    dst2_r = jnp.concatenate([dst, neg_edge_index[1], epad2]).reshape(
        NW, PIR, 128)
    par = jnp.stack([Wp2[:, 0], jnp.zeros((FD,), f32).at[0].set(bp2[0])])
    bpb = bp1[None, :]
    scores = _pred(u, v, src2_r, dst2_r, par, bpb)
    return scores[:EE, None], scores[EE:2 * EE, None], h[:NN]
